# baseline (speedup 1.0000x reference)
import functools
import numpy as np
import jax
import jax.numpy as jnp
from jax import lax
from jax.experimental import pallas as pl
from jax.experimental.pallas import tpu as pltpu
from jax.experimental.pallas import tpu_sc as plsc

_ALPHA = 4


def kernel(frames):
    B, T, C, H, W = frames.shape
    nsel = T // _ALPHA
    idx = [int(v) for v in np.linspace(0.0, T - 1, nsel).astype(np.int32)]

    info = plsc.get_sparse_core_info()
    NW = info.num_cores * info.num_subcores
    n_units = B * nsel * C // NW

    def static_lookup(table, i):
        v = jnp.int32(0)
        for j, t in enumerate(table):
            v = v + jnp.where(i == j, t, 0)
        return v

    mesh = plsc.VectorSubcoreMesh(core_axis_name="c", subcore_axis_name="s")

    @functools.partial(
        pl.kernel,
        mesh=mesh,
        out_type=jax.ShapeDtypeStruct((B, nsel, C, H, W), frames.dtype),
        scratch_types=[
            pltpu.SemaphoreType.DMA,
        ],
    )
    def gather_k(frames_hbm, slow_hbm, sem):
        wid = lax.axis_index("s") * info.num_cores + lax.axis_index("c")

        def unit(i):
            u = wid * n_units + i
            c = u % C
            s = (u // C) % nsel
            b = u // (C * nsel)
            f = static_lookup(idx, s)
            return frames_hbm.at[b, f, c], slow_hbm.at[b, s, c]

        copies = []
        for i in range(n_units):
            src, dst = unit(i)
            copies.append(pltpu.async_copy(src, dst, sem))
        for cp in copies:
            cp.wait()

    slow = gather_k(frames)

    def copy_body(x_ref, fast_ref):
        fast_ref[...] = x_ref[...]

    TB = 2
    blk = (B, TB, C, H, W)
    fast = pl.pallas_call(
        copy_body,
        grid=(T // TB,),
        in_specs=[pl.BlockSpec(blk, lambda g: (0, g, 0, 0, 0))],
        out_specs=pl.BlockSpec(blk, lambda g: (0, g, 0, 0, 0)),
        out_shape=jax.ShapeDtypeStruct((B, T, C, H, W), frames.dtype),
    )(frames)
    return (slow, fast)


# speedup vs baseline: 8.8015x; 8.8015x over previous
import functools
import numpy as np
import jax
import jax.numpy as jnp
from jax import lax
from jax.experimental import pallas as pl
from jax.experimental.pallas import tpu as pltpu
from jax.experimental.pallas import tpu_sc as plsc

_ALPHA = 4


def kernel(frames):
    B, T, C, H, W = frames.shape
    nsel = T // _ALPHA
    idx = [int(v) for v in np.linspace(0.0, T - 1, nsel).astype(np.int32)]

    info = plsc.get_sparse_core_info()
    NW = info.num_cores * info.num_subcores
    n_units = B * nsel * C // NW

    def static_lookup(table, i):
        v = jnp.int32(0)
        for j, t in enumerate(table):
            v = v + jnp.where(i == j, t, 0)
        return v

    mesh = plsc.VectorSubcoreMesh(core_axis_name="c", subcore_axis_name="s")

    @functools.partial(
        pl.kernel,
        mesh=mesh,
        out_type=jax.ShapeDtypeStruct((B, nsel, C, H, W), frames.dtype),
        scratch_types=[
            pltpu.VMEM((H, W), frames.dtype),
            pltpu.VMEM((H, W), frames.dtype),
            pltpu.SemaphoreType.DMA,
            pltpu.SemaphoreType.DMA,
            pltpu.SemaphoreType.DMA,
            pltpu.SemaphoreType.DMA,
        ],
    )
    def gather_k(frames_hbm, slow_hbm, buf0, buf1, in0, in1, out0, out1):
        wid = lax.axis_index("s") * info.num_cores + lax.axis_index("c")
        bufs, in_sems, out_sems = (buf0, buf1), (in0, in1), (out0, out1)

        def unit(i):
            u = wid * n_units + i
            c = u % C
            s = (u // C) % nsel
            b = u // (C * nsel)
            f = static_lookup(idx, s)
            return frames_hbm.at[b, f, c], slow_hbm.at[b, s, c]

        prev_out = None
        cur_in = pltpu.async_copy(unit(0)[0], bufs[0], in_sems[0])
        for i in range(n_units):
            bi = i % 2
            cur_in.wait()
            out_i = pltpu.async_copy(bufs[bi], unit(i)[1], out_sems[bi])
            if i + 1 < n_units:
                if prev_out is not None:
                    prev_out.wait()
                cur_in = pltpu.async_copy(
                    unit(i + 1)[0], bufs[(i + 1) % 2], in_sems[(i + 1) % 2])
            prev_out = out_i
        prev_out.wait()

    slow = gather_k(frames)

    def copy_body(x_ref, fast_ref):
        fast_ref[...] = x_ref[...]

    TB = 2
    blk = (B, TB, C, H, W)
    fast = pl.pallas_call(
        copy_body,
        grid=(T // TB,),
        in_specs=[pl.BlockSpec(blk, lambda g: (0, g, 0, 0, 0))],
        out_specs=pl.BlockSpec(blk, lambda g: (0, g, 0, 0, 0)),
        out_shape=jax.ShapeDtypeStruct((B, T, C, H, W), frames.dtype),
    )(frames)
    return (slow, fast)


# speedup vs baseline: 8.8097x; 1.0009x over previous
import functools
import numpy as np
import jax
import jax.numpy as jnp
from jax import lax
from jax.experimental import pallas as pl
from jax.experimental.pallas import tpu as pltpu
from jax.experimental.pallas import tpu_sc as plsc

_ALPHA = 4


def kernel(frames):
    B, T, C, H, W = frames.shape
    nsel = T // _ALPHA
    idx = [int(v) for v in np.linspace(0.0, T - 1, nsel).astype(np.int32)]

    info = plsc.get_sparse_core_info()
    NW = info.num_cores * info.num_subcores
    n_units = B * nsel * C // NW

    def static_lookup(table, i):
        v = jnp.int32(0)
        for j, t in enumerate(table):
            v = v + jnp.where(i == j, t, 0)
        return v

    mesh = plsc.VectorSubcoreMesh(core_axis_name="c", subcore_axis_name="s")

    @functools.partial(
        pl.kernel,
        mesh=mesh,
        out_type=jax.ShapeDtypeStruct((B, nsel, C, H, W), frames.dtype),
        scratch_types=[
            pltpu.VMEM((H, W), frames.dtype),
            pltpu.VMEM((H, W), frames.dtype),
            pltpu.SemaphoreType.DMA,
            pltpu.SemaphoreType.DMA,
            pltpu.SemaphoreType.DMA,
            pltpu.SemaphoreType.DMA,
        ],
    )
    def gather_k(frames_hbm, slow_hbm, buf0, buf1, in0, in1, out0, out1):
        wid = lax.axis_index("s") * info.num_cores + lax.axis_index("c")
        bufs, in_sems, out_sems = (buf0, buf1), (in0, in1), (out0, out1)

        def unit(i):
            u = wid * n_units + i
            c = u % C
            s = (u // C) % nsel
            b = u // (C * nsel)
            f = static_lookup(idx, s)
            return frames_hbm.at[b, f, c], slow_hbm.at[b, s, c]

        prev_out = None
        cur_in = pltpu.async_copy(unit(0)[0], bufs[0], in_sems[0])
        for i in range(n_units):
            bi = i % 2
            cur_in.wait()
            out_i = pltpu.async_copy(bufs[bi], unit(i)[1], out_sems[bi])
            if prev_out is not None:
                prev_out.wait()
            if i + 1 < n_units:
                cur_in = pltpu.async_copy(
                    unit(i + 1)[0], bufs[(i + 1) % 2], in_sems[(i + 1) % 2])
            prev_out = out_i
        prev_out.wait()

    slow = gather_k(frames)

    def copy_body(x_ref, fast_ref):
        fast_ref[...] = x_ref[...]

    TB = 2
    blk = (B, TB, C, H, W)
    fast = pl.pallas_call(
        copy_body,
        grid=(T // TB,),
        in_specs=[pl.BlockSpec(blk, lambda g: (0, g, 0, 0, 0))],
        out_specs=pl.BlockSpec(blk, lambda g: (0, g, 0, 0, 0)),
        out_shape=jax.ShapeDtypeStruct((B, T, C, H, W), frames.dtype),
    )(frames)
    return (slow, fast)


# speedup vs baseline: 8.8166x; 1.0008x over previous
import functools
import numpy as np
import jax
import jax.numpy as jnp
from jax import lax
from jax.experimental import pallas as pl
from jax.experimental.pallas import tpu as pltpu
from jax.experimental.pallas import tpu_sc as plsc

_ALPHA = 4


def kernel(frames):
    B, T, C, H, W = frames.shape
    nsel = T // _ALPHA
    idx = [int(v) for v in np.linspace(0.0, T - 1, nsel).astype(np.int32)]

    info = plsc.get_sparse_core_info()
    NW = info.num_cores * info.num_subcores
    n_units = B * nsel * C // NW

    def static_lookup(table, i):
        v = jnp.int32(0)
        for j, t in enumerate(table):
            v = v + jnp.where(i == j, t, 0)
        return v

    mesh = plsc.VectorSubcoreMesh(core_axis_name="c", subcore_axis_name="s")

    @functools.partial(
        pl.kernel,
        mesh=mesh,
        out_type=jax.ShapeDtypeStruct((B, nsel, C, H, W), frames.dtype),
        scratch_types=[
            pltpu.VMEM((H, W), frames.dtype),
            pltpu.VMEM((H, W), frames.dtype),
            pltpu.SemaphoreType.DMA,
            pltpu.SemaphoreType.DMA,
            pltpu.SemaphoreType.DMA,
            pltpu.SemaphoreType.DMA,
        ],
    )
    def gather_k(frames_hbm, slow_hbm, buf0, buf1, in0, in1, out0, out1):
        wid = lax.axis_index("s") * info.num_cores + lax.axis_index("c")
        bufs, in_sems, out_sems = (buf0, buf1), (in0, in1), (out0, out1)

        def unit(i):
            u = wid * n_units + i
            c = u % C
            s = (u // C) % nsel
            b = u // (C * nsel)
            f = static_lookup(idx, s)
            return frames_hbm.at[b, f, c], slow_hbm.at[b, s, c]

        for i in range(n_units):
            bi = i % 2
            src, dst = unit(i)
            pltpu.async_copy(src, bufs[bi], in_sems[bi]).wait()
            pltpu.async_copy(bufs[bi], dst, out_sems[bi]).wait()

    slow = gather_k(frames)

    def copy_body(x_ref, fast_ref):
        fast_ref[...] = x_ref[...]

    TB = 2
    blk = (B, TB, C, H, W)
    fast = pl.pallas_call(
        copy_body,
        grid=(T // TB,),
        in_specs=[pl.BlockSpec(blk, lambda g: (0, g, 0, 0, 0))],
        out_specs=pl.BlockSpec(blk, lambda g: (0, g, 0, 0, 0)),
        out_shape=jax.ShapeDtypeStruct((B, T, C, H, W), frames.dtype),
    )(frames)
    return (slow, fast)
